# baseline (speedup 1.0000x reference)
import functools

import jax
import jax.numpy as jnp
from jax import lax
from jax.experimental import pallas as pl
from jax.experimental.pallas import tpu as pltpu
from jax.experimental.pallas import tpu_sc as plsc

NC = 2
NS = 16
NW = NC * NS
L = 16

BATCH = 16384
DIM = 32
BPW = BATCH // NW
CHUNK = 128
NCHUNK = BPW // CHUNK
NGRP = BPW // L

_PARAMS = pltpu.CompilerParams(
    needs_layout_passes=False, use_tc_tiling_on_sc=False)


def _body(users_r, items_r, ut_t, it_t, out_hbm,
          uidx_v, iidx_v, ucols_v, icols_v, out_v, sem):
    wid = lax.axis_index("s") * NC + lax.axis_index("c")
    pltpu.sync_copy(users_r.at[wid], uidx_v)
    pltpu.sync_copy(items_r.at[wid], iidx_v)

    copies = []
    for c in range(DIM):
        for j in range(NCHUNK):
            copies.append(pltpu.async_copy(
                ut_t.at[c].at[uidx_v.at[j]],
                ucols_v.at[c, pl.ds(j * CHUNK, CHUNK)], sem))
            copies.append(pltpu.async_copy(
                it_t.at[c].at[iidx_v.at[j]],
                icols_v.at[c, pl.ds(j * CHUNK, CHUNK)], sem))
    for cp in copies:
        cp.wait()

    def grp(g, carry):
        acc = jnp.zeros((L,), jnp.float32)
        for c in range(DIM):
            u = ucols_v[c, pl.ds(g * L, L)]
            v = icols_v[c, pl.ds(g * L, L)]
            acc = acc + u * v
        rating = 1.0 / (1.0 + jnp.exp(-acc))
        out_v[pl.ds(g * L, L)] = rating
        return carry

    lax.fori_loop(0, NGRP, grp, 0)
    pltpu.sync_copy(out_v, out_hbm.at[wid])


@jax.jit
def _run(users_r, items_r, ut_t, it_t):
    mesh = plsc.VectorSubcoreMesh(core_axis_name="c", subcore_axis_name="s")
    f = pl.kernel(
        _body,
        out_type=jax.ShapeDtypeStruct((NW, BPW), jnp.float32),
        mesh=mesh,
        scratch_types=[
            pltpu.VMEM((NCHUNK, CHUNK), jnp.int32),
            pltpu.VMEM((NCHUNK, CHUNK), jnp.int32),
            pltpu.VMEM((DIM, BPW), jnp.float32),
            pltpu.VMEM((DIM, BPW), jnp.float32),
            pltpu.VMEM((BPW,), jnp.float32),
            pltpu.SemaphoreType.DMA,
        ],
        compiler_params=_PARAMS,
    )
    return f(users_r, items_r, ut_t, it_t)


def kernel(users, items, user_table, item_table):
    users_r = users.reshape(NW, NCHUNK, CHUNK)
    items_r = items.reshape(NW, NCHUNK, CHUNK)
    out = _run(users_r, items_r, user_table.T, item_table.T)
    return out.reshape(BATCH)


# speedup vs baseline: 5.4942x; 5.4942x over previous
import jax
import jax.numpy as jnp
from jax import lax
from jax.experimental import pallas as pl
from jax.experimental.pallas import tpu as pltpu
from jax.experimental.pallas import tpu_sc as plsc

NC = 2
NS = 16
NW = NC * NS
BATCH = 16384
DIM = 32
BPW = BATCH // NW
CHUNK = 128
NCHUNK = BPW // CHUNK

_PARAMS = pltpu.CompilerParams(
    needs_layout_passes=False, use_tc_tiling_on_sc=False)


def _gather_body(users_r, items_r, ut, it, out_hbm,
                 uidx_v, iidx_v, urows_v, irows_v, sem):
    wid = lax.axis_index("s") * NC + lax.axis_index("c")
    pltpu.sync_copy(users_r.at[wid], uidx_v)
    pltpu.sync_copy(items_r.at[wid], iidx_v)

    copies = []
    for j in range(NCHUNK):
        copies.append(pltpu.async_copy(
            ut.at[uidx_v.at[j]], urows_v.at[pl.ds(j * CHUNK, CHUNK)], sem))
        copies.append(pltpu.async_copy(
            it.at[iidx_v.at[j]], irows_v.at[pl.ds(j * CHUNK, CHUNK)], sem))
    for cp in copies:
        cp.wait()

    pltpu.sync_copy(urows_v, out_hbm.at[0, wid])
    pltpu.sync_copy(irows_v, out_hbm.at[1, wid])


@jax.jit
def _run(users_r, items_r, ut, it):
    mesh = plsc.VectorSubcoreMesh(core_axis_name="c", subcore_axis_name="s")
    gather = pl.kernel(
        _gather_body,
        out_type=jax.ShapeDtypeStruct((2, NW, BPW, DIM), jnp.float32),
        mesh=mesh,
        scratch_types=[
            pltpu.VMEM((NCHUNK, CHUNK), jnp.int32),
            pltpu.VMEM((NCHUNK, CHUNK), jnp.int32),
            pltpu.VMEM((BPW, DIM), jnp.float32),
            pltpu.VMEM((BPW, DIM), jnp.float32),
            pltpu.SemaphoreType.DMA,
        ],
        compiler_params=_PARAMS,
    )
    rows = gather(users_r, items_r, ut, it)
    u_e = rows[0].reshape(BATCH, DIM)
    i_e = rows[1].reshape(BATCH, DIM)

    def dot_body(u_ref, i_ref, o_ref):
        o_ref[...] = jax.nn.sigmoid(
            jnp.sum(u_ref[...] * i_ref[...], axis=1))

    return pl.pallas_call(
        dot_body,
        out_shape=jax.ShapeDtypeStruct((BATCH,), jnp.float32),
    )(u_e, i_e)


def kernel(users, items, user_table, item_table):
    users_r = users.reshape(NW, NCHUNK, CHUNK)
    items_r = items.reshape(NW, NCHUNK, CHUNK)
    return _run(users_r, items_r, user_table, item_table)


# speedup vs baseline: 5.5975x; 1.0188x over previous
import jax
import jax.numpy as jnp
from jax import lax
from jax.experimental import pallas as pl
from jax.experimental.pallas import tpu as pltpu
from jax.experimental.pallas import tpu_sc as plsc

NC = 2
NS = 16
NW = NC * NS
L = 16
BATCH = 16384
DIM = 32
PACK = 128 // DIM
BPW = BATCH // NW
CHUNK = 128
NCHUNK = BPW // CHUNK
HALF = BPW // 2
NGRP = HALF // L

_PARAMS = pltpu.CompilerParams(
    needs_layout_passes=False, use_tc_tiling_on_sc=False)


def _body(rpu_r, rpi_r, offu_r, offi_r, pu, pi, out_hbm,
          uidx_v, iidx_v, offu_v, offi_v, urows_v, irows_v, out_v, sem):
    wid = lax.axis_index("s") * NC + lax.axis_index("c")
    pltpu.sync_copy(rpu_r.at[wid], uidx_v)
    pltpu.sync_copy(rpi_r.at[wid], iidx_v)
    pltpu.sync_copy(offu_r.at[wid], offu_v)
    pltpu.sync_copy(offi_r.at[wid], offi_v)

    iota = lax.iota(jnp.int32, L)

    for h in range(2):
        copies = []
        for j in range(NCHUNK // 2):
            copies.append(pltpu.async_copy(
                pu.at[uidx_v.at[2 * h + j]],
                urows_v.at[pl.ds(j * CHUNK, CHUNK)], sem))
            copies.append(pltpu.async_copy(
                pi.at[iidx_v.at[2 * h + j]],
                irows_v.at[pl.ds(j * CHUNK, CHUNK)], sem))
        for cp in copies:
            cp.wait()

        def grp(g, carry):
            base = h * HALF + g * L
            rows = iota + g * L
            ou = offu_v[pl.ds(base, L)]
            oi = offi_v[pl.ds(base, L)]
            acc = jnp.zeros((L,), jnp.float32)
            for d in range(DIM):
                uv = plsc.load_gather(urows_v, [rows, ou + d])
                iv = plsc.load_gather(irows_v, [rows, oi + d])
                acc = acc + uv * iv
            out_v[pl.ds(base, L)] = 1.0 / (1.0 + jnp.exp(-acc))
            return carry

        lax.fori_loop(0, NGRP, grp, 0)

    pltpu.sync_copy(out_v, out_hbm.at[wid])


@jax.jit
def _run(rpu_r, rpi_r, offu_r, offi_r, pu, pi):
    mesh = plsc.VectorSubcoreMesh(core_axis_name="c", subcore_axis_name="s")
    f = pl.kernel(
        _body,
        out_type=jax.ShapeDtypeStruct((NW, BPW), jnp.float32),
        mesh=mesh,
        scratch_types=[
            pltpu.VMEM((NCHUNK, CHUNK), jnp.int32),
            pltpu.VMEM((NCHUNK, CHUNK), jnp.int32),
            pltpu.VMEM((BPW,), jnp.int32),
            pltpu.VMEM((BPW,), jnp.int32),
            pltpu.VMEM((HALF, 128), jnp.float32),
            pltpu.VMEM((HALF, 128), jnp.float32),
            pltpu.VMEM((BPW,), jnp.float32),
            pltpu.SemaphoreType.DMA,
        ],
        compiler_params=_PARAMS,
    )
    return f(rpu_r, rpi_r, offu_r, offi_r, pu, pi)


def kernel(users, items, user_table, item_table):
    pu = user_table.reshape(1000000 // PACK, DIM * PACK)
    pi = item_table.reshape(1000000 // PACK, DIM * PACK)
    rpu = (users // PACK).reshape(NW, NCHUNK, CHUNK)
    rpi = (items // PACK).reshape(NW, NCHUNK, CHUNK)
    offu = ((users % PACK) * DIM).reshape(NW, BPW)
    offi = ((items % PACK) * DIM).reshape(NW, BPW)
    out = _run(rpu, rpi, offu, offi, pu, pi)
    return out.reshape(BATCH)
